# baseline (speedup 1.0000x reference)
import functools

import jax
import jax.numpy as jnp
from jax import lax
from jax.experimental import pallas as pl
from jax.experimental.pallas import tpu as pltpu
from jax.experimental.pallas import tpu_sc as plsc

_NC = 2
_NS = 16
_NW = _NC * _NS
_SUB = 128
_NBUF = 2


@functools.cache
def _make_gather(bsz, hist, emb):
    rows_per_w = bsz // _NW
    nblk = rows_per_w // _NBUF
    ib = _NBUF * hist
    mesh = plsc.VectorSubcoreMesh(core_axis_name="c", subcore_axis_name="s")

    @functools.partial(
        pl.kernel,
        mesh=mesh,
        out_type=jax.ShapeDtypeStruct((bsz, hist, emb), jnp.float32),
        compiler_params=pltpu.CompilerParams(use_tc_tiling_on_sc=False),
        scratch_types=[
            pltpu.VMEM((ib,), jnp.int32),
            pltpu.VMEM((_NBUF, hist, emb), jnp.float32),
            pltpu.SemaphoreType.DMA,
            pltpu.SemaphoreType.DMA,
            pltpu.SemaphoreType.DMA,
        ],
    )
    def gather(ids_hbm, table_hbm, out_hbm, idx_v, rows_v, gsem, osem0,
               osem1):
        wid = lax.axis_index("s") * _NC + lax.axis_index("c")
        base_row = wid * rows_per_w
        osems = (osem0, osem1)
        nsub = (hist + _SUB - 1) // _SUB

        def body(blk, carry):
            bi = base_row + blk * _NBUF
            pltpu.sync_copy(ids_hbm.at[pl.ds(bi * hist, ib)], idx_v)
            for b in range(_NBUF):
                @pl.when(blk > 0)
                def _():
                    pltpu.make_async_copy(
                        rows_v.at[b],
                        out_hbm.at[base_row],
                        osems[b],
                    ).wait()

                handles = []
                for j in range(nsub):
                    off = j * _SUB
                    sz = min(_SUB, hist - off)
                    handles.append(pltpu.async_copy(
                        table_hbm.at[idx_v.at[pl.ds(b * hist + off, sz)]],
                        rows_v.at[b].at[pl.ds(off, sz)],
                        gsem,
                    ))
                for h in handles:
                    h.wait()
                pltpu.async_copy(
                    rows_v.at[b],
                    out_hbm.at[bi + b],
                    osems[b],
                )
            return carry

        lax.fori_loop(0, nblk, body, 0)
        for b in range(_NBUF):
            pltpu.make_async_copy(
                rows_v.at[b],
                out_hbm.at[base_row],
                osems[b],
            ).wait()

    return gather


def kernel(title_ids, title_embedding):
    b, h = title_ids.shape
    emb = title_embedding.shape[1]
    ids = title_ids.reshape(-1).astype(jnp.int32)
    return _make_gather(b, h, emb)(ids, title_embedding)


# speedup vs baseline: 1.7705x; 1.7705x over previous
import functools

import jax
import jax.numpy as jnp
from jax import lax
from jax.experimental import pallas as pl
from jax.experimental.pallas import tpu as pltpu
from jax.experimental.pallas import tpu_sc as plsc

_NC = 2
_NS = 16
_NW = _NC * _NS
_BG = 128
_SL = 8
_STRIDE = 65
_NBUF = 2


@functools.cache
def _make_gather(bsz, hist, emb, vocab):
    ngrp = bsz // _BG
    grp_per_w = ngrp // _NW
    dt = emb // _SL
    mesh = plsc.VectorSubcoreMesh(core_axis_name="c", subcore_axis_name="s")

    @functools.partial(
        pl.kernel,
        mesh=mesh,
        out_type=jax.ShapeDtypeStruct((hist, dt, ngrp, _SL, _BG),
                                      jnp.float32),
        compiler_params=pltpu.CompilerParams(use_tc_tiling_on_sc=False,
                                             needs_layout_passes=False),
        scratch_types=[
            pltpu.VMEM((vocab * _STRIDE,), jnp.float32),
            pltpu.VMEM((_NBUF, dt, _SL, _BG), jnp.float32),
            pltpu.VMEM((hist, _BG), jnp.int32),
            pltpu.SemaphoreType.DMA,
            pltpu.SemaphoreType.DMA,
        ],
    )
    def gather(ids_hbm, table_hbm, out_hbm, table_v, out_v, idx_v, osem0,
               osem1):
        wid = lax.axis_index("s") * _NC + lax.axis_index("c")
        base_bg = wid * grp_per_w
        osems = (osem0, osem1)

        pltpu.sync_copy(table_hbm, table_v)

        def do_unit(h, bg, b, first):
            if not first:
                pltpu.make_async_copy(
                    out_v.at[b],
                    out_hbm.at[0, :, 0],
                    osems[b],
                ).wait()
            def gloop(g, carry):
                g16 = g * 16
                idxg = idx_v[h, pl.ds(g16, 16)] * _STRIDE
                for d8 in range(dt):
                    for dd in range(_SL):
                        out_v[b, d8, dd, pl.ds(g16, 16)] = (
                            plsc.load_gather(table_v,
                                             [idxg + (d8 * _SL + dd)]))
                return carry

            lax.fori_loop(0, _BG // 16, gloop, 0)
            pltpu.async_copy(out_v.at[b], out_hbm.at[h, :, bg], osems[b])

        for bgi in range(grp_per_w):
            bg = base_bg + bgi
            pltpu.sync_copy(ids_hbm.at[:, pl.ds(bg * _BG, _BG)], idx_v)

            if bgi == 0:
                do_unit(0, bg, 0, True)
                do_unit(1, bg, 1, True)

                def body0(h2, carry):
                    do_unit(h2 * 2, bg, 0, False)
                    do_unit(h2 * 2 + 1, bg, 1, False)
                    return carry

                lax.fori_loop(1, hist // 2, body0, 0)
            else:
                def body(h2, carry):
                    do_unit(h2 * 2, bg, 0, False)
                    do_unit(h2 * 2 + 1, bg, 1, False)
                    return carry

                lax.fori_loop(0, hist // 2, body, 0)

        for b in range(_NBUF):
            pltpu.make_async_copy(
                out_v.at[b],
                out_hbm.at[0, :, 0],
                osems[b],
            ).wait()

    return gather


def kernel(title_ids, title_embedding):
    b, h = title_ids.shape
    vocab, emb = title_embedding.shape
    ids_t = title_ids.T.astype(jnp.int32)
    table = jnp.pad(title_embedding,
                    ((0, 0), (0, _STRIDE - emb))).reshape(-1)
    out5 = _make_gather(b, h, emb, vocab)(ids_t, table)
    return jnp.transpose(out5, (2, 4, 0, 1, 3)).reshape(b, h, emb)


# speedup vs baseline: 4.2323x; 2.3905x over previous
import functools

import jax
import jax.numpy as jnp
from jax import lax
from jax.experimental import pallas as pl
from jax.experimental.pallas import tpu as pltpu
from jax.experimental.pallas import tpu_sc as plsc

_NC = 2
_NS = 16
_NW = _NC * _NS
_BG = 128
_SL = 8
_STRIDE = 65
_NBUF = 2


@functools.cache
def _make_gather(bsz, hist, emb, vocab):
    ngrp = bsz // _BG
    grp_per_w = ngrp // _NW
    dt = emb // _SL
    mesh = plsc.VectorSubcoreMesh(core_axis_name="c", subcore_axis_name="s")

    @functools.partial(
        pl.kernel,
        mesh=mesh,
        out_type=jax.ShapeDtypeStruct((hist, dt, ngrp, _SL, _BG),
                                      jnp.float32),
        compiler_params=pltpu.CompilerParams(use_tc_tiling_on_sc=False,
                                             needs_layout_passes=False),
        scratch_types=[
            pltpu.VMEM((vocab * _STRIDE,), jnp.float32),
            pltpu.VMEM((_NBUF, dt, _SL, _BG), jnp.float32),
            pltpu.VMEM((hist, _BG), jnp.int32),
            pltpu.SemaphoreType.DMA,
            pltpu.SemaphoreType.DMA,
        ],
    )
    def gather(ids_hbm, table_hbm, out_hbm, table_v, out_v, idx_v, osem0,
               osem1):
        wid = lax.axis_index("s") * _NC + lax.axis_index("c")
        base_bg = wid * grp_per_w
        osems = (osem0, osem1)

        pltpu.sync_copy(table_hbm, table_v)

        def do_unit(h, bg, b, first):
            if not first:
                pltpu.make_async_copy(
                    out_v.at[b],
                    out_hbm.at[0, :, 0],
                    osems[b],
                ).wait()
            nlane = 4

            @plsc.parallel_loop(0, _BG // (16 * nlane))
            def gloop(g):
                offs = [g * 16 * nlane + k * 16 for k in range(nlane)]
                idxs = [idx_v[h, pl.ds(o, 16)] * _STRIDE for o in offs]
                for d8 in range(dt):
                    for dd in range(_SL):
                        c = d8 * _SL + dd
                        vals = [plsc.load_gather(table_v, [ix + c])
                                for ix in idxs]
                        for o, v in zip(offs, vals):
                            out_v[b, d8, dd, pl.ds(o, 16)] = v
            pltpu.async_copy(out_v.at[b], out_hbm.at[h, :, bg], osems[b])

        for bgi in range(grp_per_w):
            bg = base_bg + bgi
            pltpu.sync_copy(ids_hbm.at[:, pl.ds(bg * _BG, _BG)], idx_v)

            if bgi == 0:
                do_unit(0, bg, 0, True)
                do_unit(1, bg, 1, True)

                def body0(h2, carry):
                    do_unit(h2 * 2, bg, 0, False)
                    do_unit(h2 * 2 + 1, bg, 1, False)
                    return carry

                lax.fori_loop(1, hist // 2, body0, 0)
            else:
                def body(h2, carry):
                    do_unit(h2 * 2, bg, 0, False)
                    do_unit(h2 * 2 + 1, bg, 1, False)
                    return carry

                lax.fori_loop(0, hist // 2, body, 0)

        for b in range(_NBUF):
            pltpu.make_async_copy(
                out_v.at[b],
                out_hbm.at[0, :, 0],
                osems[b],
            ).wait()

    return gather


def kernel(title_ids, title_embedding):
    b, h = title_ids.shape
    vocab, emb = title_embedding.shape
    ids_t = title_ids.T.astype(jnp.int32)
    table = jnp.pad(title_embedding,
                    ((0, 0), (0, _STRIDE - emb))).reshape(-1)
    out5 = _make_gather(b, h, emb, vocab)(ids_t, table)
    return jnp.transpose(out5, (2, 4, 0, 1, 3)).reshape(b, h, emb)
